# Initial kernel scaffold; baseline (speedup 1.0000x reference)
import jax
import jax.numpy as jnp
from jax.experimental import pallas as pl


def kernel(node_input, embeddings, mask_idx, struct_target, svd_target, mask_token, W1, b1, W2, b2):
    raise NotImplementedError("write your pallas kernel here")


# speedup vs baseline: 1.6415x; 1.6415x over previous
import functools

import jax
import jax.numpy as jnp
from jax import lax
from jax.experimental import pallas as pl
from jax.experimental.pallas import tpu as pltpu
from jax.experimental.pallas import tpu_sc as plsc

N, D = 50000, 512
M = 7500
SD, VD = 6, 256
SDP = 128

NC, NS = 2, 16
NW = NC * NS
MP = 7680
BPW = MP // NW
CH = 48
NCH = BPW // CH
NPAD = 53248
FPT = NPAD // NW

BM = 512
RSEL = 512


def _sc_body(idx_hbm, emb_hbm, svd_hbm, strp_hbm,
             f_hbm, h_hbm, svdg_hbm, strg_hbm,
             idx_all, fseg, emb_v, svd_v, str_v, sem):
    c = lax.axis_index("c")
    s = lax.axis_index("s")
    wid = s * NC + c
    lo = wid * FPT

    pltpu.sync_copy(idx_hbm, idx_all)

    zf = jnp.zeros((16,), jnp.float32)

    def zbody(i, carry):
        fseg[pl.ds(i * 16, 16)] = zf
        return carry

    lax.fori_loop(0, FPT // 16, zbody, 0)

    ones = jnp.full((16,), 1.0, jnp.float32)

    def obody(i, carry):
        v = idx_all[pl.ds(i * 16, 16)]
        m = (v >= lo) & (v < lo + FPT)
        plsc.store_scatter(fseg, [v - lo], ones, mask=m)
        return carry

    lax.fori_loop(0, MP // 16, obody, 0)
    pltpu.sync_copy(fseg, f_hbm.at[0, pl.ds(lo, FPT)])

    base = wid * BPW
    for j in range(NCH):
        idx_c = idx_all.at[pl.ds(base + j * CH, CH)]
        out = pl.ds(base + j * CH, CH)
        pltpu.async_copy(emb_hbm.at[idx_c], emb_v, sem).wait()
        pltpu.sync_copy(emb_v, h_hbm.at[out])
        pltpu.async_copy(svd_hbm.at[idx_c], svd_v, sem).wait()
        pltpu.sync_copy(svd_v, svdg_hbm.at[out])
        pltpu.async_copy(strp_hbm.at[idx_c], str_v, sem).wait()
        pltpu.sync_copy(str_v, strg_hbm.at[out])


_sc_call = functools.partial(
    pl.kernel,
    out_type=(
        jax.ShapeDtypeStruct((1, NPAD), jnp.float32),
        jax.ShapeDtypeStruct((MP, D), jnp.float32),
        jax.ShapeDtypeStruct((MP, VD), jnp.float32),
        jax.ShapeDtypeStruct((MP, SDP), jnp.float32),
    ),
    mesh=plsc.VectorSubcoreMesh(
        core_axis_name="c", subcore_axis_name="s",
        num_cores=NC, num_subcores=NS),
    compiler_params=pltpu.CompilerParams(needs_layout_passes=False),
    scratch_types=[
        pltpu.VMEM((MP,), jnp.int32),
        pltpu.VMEM((FPT,), jnp.float32),
        pltpu.VMEM((CH, D), jnp.float32),
        pltpu.VMEM((CH, VD), jnp.float32),
        pltpu.VMEM((CH, SDP), jnp.float32),
        pltpu.SemaphoreType.DMA,
    ],
)(_sc_body)


def _sel_body(node_ref, f_ref, tok_ref, out_ref):
    flag = f_ref[...]
    ri = lax.broadcasted_iota(jnp.int32, (RSEL, RSEL), 0)
    ci = lax.broadcasted_iota(jnp.int32, (RSEL, RSEL), 1)
    fcol = jnp.sum(jnp.where(ri == ci, flag, 0.0), axis=1, keepdims=True)
    out_ref[...] = jnp.where(fcol > 0.0, tok_ref[...], node_ref[...])


def _select(node_input, f, mask_token):
    return pl.pallas_call(
        _sel_body,
        grid=(pl.cdiv(N, RSEL),),
        in_specs=[
            pl.BlockSpec((RSEL, D), lambda i: (i, 0)),
            pl.BlockSpec((1, RSEL), lambda i: (0, i)),
            pl.BlockSpec((1, D), lambda i: (0, 0)),
        ],
        out_specs=pl.BlockSpec((RSEL, D), lambda i: (i, 0)),
        out_shape=jax.ShapeDtypeStruct((N, D), jnp.float32),
    )(node_input, f, mask_token)


def _loss_body(h_ref, w1_ref, b1_ref, w2s_ref, b2s_ref, w2v_ref, b2v_ref,
               strg_ref, svdg_ref, out_ref):
    i = pl.program_id(0)

    h1 = jnp.dot(h_ref[...], w1_ref[...], preferred_element_type=jnp.float32)
    h1 = jnp.maximum(h1 + b1_ref[...], 0.0)
    pred_s = jnp.dot(h1, w2s_ref[...], preferred_element_type=jnp.float32)
    pred_s = pred_s + b2s_ref[...]
    pred_v = jnp.dot(h1, w2v_ref[...], preferred_element_type=jnp.float32)
    pred_v = pred_v + b2v_ref[...]

    ds_ = pred_s - strg_ref[...]
    dv_ = pred_v - svdg_ref[...]
    rows = i * BM + lax.broadcasted_iota(jnp.int32, (BM, 1), 0)
    valid = rows < M
    part = (jnp.sum(jnp.where(valid, ds_ * ds_, 0.0))
            + jnp.sum(jnp.where(valid, dv_ * dv_, 0.0)))

    @pl.when(i == 0)
    def _():
        out_ref[0, 0] = 0.0

    out_ref[0, 0] += part

    @pl.when(i == pl.num_programs(0) - 1)
    def _():
        out_ref[0, 0] = out_ref[0, 0] * (1.0 / (M * (SD + VD)))


def _loss(h, w1, b1, w2s, b2s, w2v, b2v, strg, svdg):
    return pl.pallas_call(
        _loss_body,
        grid=(MP // BM,),
        in_specs=[
            pl.BlockSpec((BM, D), lambda i: (i, 0)),
            pl.BlockSpec((D, D), lambda i: (0, 0)),
            pl.BlockSpec((1, D), lambda i: (0, 0)),
            pl.BlockSpec((D, SDP), lambda i: (0, 0)),
            pl.BlockSpec((1, SDP), lambda i: (0, 0)),
            pl.BlockSpec((D, VD), lambda i: (0, 0)),
            pl.BlockSpec((1, VD), lambda i: (0, 0)),
            pl.BlockSpec((BM, SDP), lambda i: (i, 0)),
            pl.BlockSpec((BM, VD), lambda i: (i, 0)),
        ],
        out_specs=pl.BlockSpec(memory_space=pltpu.SMEM),
        out_shape=jax.ShapeDtypeStruct((1, 1), jnp.float32),
    )(h, w1, b1, w2s, b2s, w2v, b2v, strg, svdg)


def kernel(node_input, embeddings, mask_idx, struct_target, svd_target,
           mask_token, W1, b1, W2, b2):
    idx = mask_idx.astype(jnp.int32)
    idx_p = jnp.concatenate([idx, jnp.broadcast_to(idx[:1], (MP - M,))])
    strp = jnp.pad(struct_target, ((0, 0), (0, SDP - SD)))

    f, h, svdg, strg = _sc_call(idx_p, embeddings, svd_target, strp)

    masked = _select(node_input, f, mask_token)


    w2s = jnp.pad(W2[:, :SD], ((0, 0), (0, SDP - SD)))
    b2s = jnp.pad(b2[:SD], (0, SDP - SD)).reshape(1, SDP)
    loss = _loss(h, W1, b1.reshape(1, D), w2s, b2s,
                 W2[:, SD:], b2[SD:].reshape(1, VD), strg, svdg)
    return masked, loss[0, 0]


# speedup vs baseline: 1.8815x; 1.1462x over previous
import functools

import jax
import jax.numpy as jnp
from jax import lax
from jax.experimental import pallas as pl
from jax.experimental.pallas import tpu as pltpu
from jax.experimental.pallas import tpu_sc as plsc

N, D = 50000, 512
M = 7500
SD, VD = 6, 256
SDP = 128

NC, NS = 2, 16
NW = NC * NS
MP = 7680
BPW = MP // NW
CH = 48
NCH = BPW // CH
NPAD = 53248
FPT = NPAD // NW

BM = 512
RSEL = 512


def _flags_body(idx_hbm, f_hbm, idx_all, fseg):
    c = lax.axis_index("c")
    s = lax.axis_index("s")
    wid = s * NC + c
    lo = wid * FPT

    pltpu.sync_copy(idx_hbm, idx_all)

    zf = jnp.zeros((16,), jnp.float32)

    def zbody(i, carry):
        fseg[pl.ds(i * 16, 16)] = zf
        return carry

    lax.fori_loop(0, FPT // 16, zbody, 0)

    ones = jnp.full((16,), 1.0, jnp.float32)

    def obody(i, carry):
        v = idx_all[pl.ds(i * 16, 16)]
        m = (v >= lo) & (v < lo + FPT)
        plsc.store_scatter(fseg, [v - lo], ones, mask=m)
        return carry

    lax.fori_loop(0, MP // 16, obody, 0)
    pltpu.sync_copy(fseg, f_hbm.at[0, pl.ds(lo, FPT)])


_flags_call = functools.partial(
    pl.kernel,
    out_type=jax.ShapeDtypeStruct((1, NPAD), jnp.float32),
    mesh=plsc.VectorSubcoreMesh(
        core_axis_name="c", subcore_axis_name="s",
        num_cores=NC, num_subcores=NS),
    compiler_params=pltpu.CompilerParams(needs_layout_passes=False),
    scratch_types=[
        pltpu.VMEM((MP,), jnp.int32),
        pltpu.VMEM((FPT,), jnp.float32),
    ],
)(_flags_body)


def _gather_body(idx_hbm, emb_hbm, svd_hbm, strp_hbm,
                 h_hbm, svdg_hbm, strg_hbm,
                 idx_v, emb_v, svd_v, str_v, gsems, wsems):
    c = lax.axis_index("c")
    s = lax.axis_index("s")
    wid = s * NC + c
    base = wid * BPW

    pltpu.sync_copy(idx_hbm.at[pl.ds(base, BPW)], idx_v)

    srcs = (emb_hbm, svd_hbm, strp_hbm)
    bufs = (emb_v, svd_v, str_v)
    outs = (h_hbm, svdg_hbm, strg_hbm)
    gdesc = [[None, None] for _ in range(3)]
    wdesc = [[None, None] for _ in range(3)]

    def start_gathers(j):
        b = j % 2
        idx_c = idx_v.at[pl.ds(j * CH, CH)]
        for a in range(3):
            gdesc[a][b] = pltpu.async_copy(
                srcs[a].at[idx_c], bufs[a].at[b], gsems.at[a, b])

    def start_writes(j):
        b = j % 2
        dst = pl.ds(base + j * CH, CH)
        for a in range(3):
            gdesc[a][b].wait()
            wdesc[a][b] = pltpu.async_copy(
                bufs[a].at[b], outs[a].at[dst], wsems.at[a, b])

    for j in range(NCH):
        b = j % 2
        if j >= 2:
            for a in range(3):
                wdesc[a][b].wait()
        start_gathers(j)
        if j >= 1:
            start_writes(j - 1)
    start_writes(NCH - 1)
    for a in range(3):
        wdesc[a][(NCH - 2) % 2].wait()
        wdesc[a][(NCH - 1) % 2].wait()


_gather_call = functools.partial(
    pl.kernel,
    out_type=(
        jax.ShapeDtypeStruct((MP, D), jnp.float32),
        jax.ShapeDtypeStruct((MP, VD), jnp.float32),
        jax.ShapeDtypeStruct((MP, SDP), jnp.float32),
    ),
    mesh=plsc.VectorSubcoreMesh(
        core_axis_name="c", subcore_axis_name="s",
        num_cores=NC, num_subcores=NS),
    compiler_params=pltpu.CompilerParams(needs_layout_passes=False),
    scratch_types=[
        pltpu.VMEM((BPW,), jnp.int32),
        pltpu.VMEM((2, CH, D), jnp.float32),
        pltpu.VMEM((2, CH, VD), jnp.float32),
        pltpu.VMEM((2, CH, SDP), jnp.float32),
        pltpu.SemaphoreType.DMA((3, 2)),
        pltpu.SemaphoreType.DMA((3, 2)),
    ],
)(_gather_body)


def _sel_body(node_ref, f_ref, tok_ref, out_ref):
    flag = f_ref[...]
    ri = lax.broadcasted_iota(jnp.int32, (RSEL, RSEL), 0)
    ci = lax.broadcasted_iota(jnp.int32, (RSEL, RSEL), 1)
    fcol = jnp.sum(jnp.where(ri == ci, flag, 0.0), axis=1, keepdims=True)
    out_ref[...] = jnp.where(fcol > 0.0, tok_ref[...], node_ref[...])


def _select(node_input, f, mask_token):
    return pl.pallas_call(
        _sel_body,
        grid=(pl.cdiv(N, RSEL),),
        in_specs=[
            pl.BlockSpec((RSEL, D), lambda i: (i, 0)),
            pl.BlockSpec((1, RSEL), lambda i: (0, i)),
            pl.BlockSpec((1, D), lambda i: (0, 0)),
        ],
        out_specs=pl.BlockSpec((RSEL, D), lambda i: (i, 0)),
        out_shape=jax.ShapeDtypeStruct((N, D), jnp.float32),
    )(node_input, f, mask_token)


def _loss_body(h_ref, w1_ref, b1_ref, w2s_ref, b2s_ref, w2v_ref, b2v_ref,
               strg_ref, svdg_ref, out_ref):
    i = pl.program_id(0)

    h1 = jnp.dot(h_ref[...], w1_ref[...], preferred_element_type=jnp.float32)
    h1 = jnp.maximum(h1 + b1_ref[...], 0.0)
    pred_s = jnp.dot(h1, w2s_ref[...], preferred_element_type=jnp.float32)
    pred_s = pred_s + b2s_ref[...]
    pred_v = jnp.dot(h1, w2v_ref[...], preferred_element_type=jnp.float32)
    pred_v = pred_v + b2v_ref[...]

    ds_ = pred_s - strg_ref[...]
    dv_ = pred_v - svdg_ref[...]
    rows = i * BM + lax.broadcasted_iota(jnp.int32, (BM, 1), 0)
    valid = rows < M
    part = (jnp.sum(jnp.where(valid, ds_ * ds_, 0.0))
            + jnp.sum(jnp.where(valid, dv_ * dv_, 0.0)))

    @pl.when(i == 0)
    def _():
        out_ref[0, 0] = 0.0

    out_ref[0, 0] += part

    @pl.when(i == pl.num_programs(0) - 1)
    def _():
        out_ref[0, 0] = out_ref[0, 0] * (1.0 / (M * (SD + VD)))


def _loss(h, w1, b1, w2s, b2s, w2v, b2v, strg, svdg):
    return pl.pallas_call(
        _loss_body,
        grid=(MP // BM,),
        in_specs=[
            pl.BlockSpec((BM, D), lambda i: (i, 0)),
            pl.BlockSpec((D, D), lambda i: (0, 0)),
            pl.BlockSpec((1, D), lambda i: (0, 0)),
            pl.BlockSpec((D, SDP), lambda i: (0, 0)),
            pl.BlockSpec((1, SDP), lambda i: (0, 0)),
            pl.BlockSpec((D, VD), lambda i: (0, 0)),
            pl.BlockSpec((1, VD), lambda i: (0, 0)),
            pl.BlockSpec((BM, SDP), lambda i: (i, 0)),
            pl.BlockSpec((BM, VD), lambda i: (i, 0)),
        ],
        out_specs=pl.BlockSpec(memory_space=pltpu.SMEM),
        out_shape=jax.ShapeDtypeStruct((1, 1), jnp.float32),
    )(h, w1, b1, w2s, b2s, w2v, b2v, strg, svdg)


def kernel(node_input, embeddings, mask_idx, struct_target, svd_target,
           mask_token, W1, b1, W2, b2):
    idx = mask_idx.astype(jnp.int32)
    idx_p = jnp.concatenate([idx, jnp.broadcast_to(idx[:1], (MP - M,))])
    strp = jnp.pad(struct_target, ((0, 0), (0, SDP - SD)))

    f = _flags_call(idx_p)
    h, svdg, strg = _gather_call(idx_p, embeddings, svd_target, strp)

    masked = _select(node_input, f, mask_token)


    w2s = jnp.pad(W2[:, :SD], ((0, 0), (0, SDP - SD)))
    b2s = jnp.pad(b2[:SD], (0, SDP - SD)).reshape(1, SDP)
    loss = _loss(h, W1, b1.reshape(1, D), w2s, b2s,
                 W2[:, SD:], b2[SD:].reshape(1, VD), strg, svdg)
    return masked, loss[0, 0]
